# baseline (speedup 1.0000x reference)
import functools

import jax
import jax.numpy as jnp
from jax import lax
from jax.experimental import pallas as pl
from jax.experimental.pallas import tpu as pltpu
from jax.experimental.pallas import tpu_sc as plsc

D = 64
SCALE = 8.0
SUB = 128
GPC = 2
CHUNK = SUB * GPC
NBUF = 4
BW = 256

_info = plsc.get_sparse_core_info()
_NC, _NS = _info.num_cores, _info.num_subcores
NW = _NC * _NS

_MESH = dict(core_axis_name="c", subcore_axis_name="s")


def _wid():
    return lax.axis_index("s") * _NC + lax.axis_index("c")


def _make_detranspose(n_rows: int):
    full_blocks = n_rows // 128
    tail = n_rows - full_blocks * 128
    per_w = (full_blocks + NW - 1) // NW

    @functools.partial(
        pl.kernel,
        out_type=jax.ShapeDtypeStruct((n_rows * D,), jnp.float32),
        mesh=plsc.VectorSubcoreMesh(**_MESH),
        scratch_types=[
            pltpu.VMEM((D, 128), jnp.float32),
            pltpu.VMEM((128 * D,), jnp.float32),
            pltpu.SemaphoreType.DMA,
        ],
        compiler_params=pltpu.CompilerParams(
            use_tc_tiling_on_sc=True, needs_layout_passes=False,
            disable_bounds_checks=True),
    )
    def detranspose(tab_t, out, src_v, dst_v, sem):
        w = _wid()
        lo = w * per_w
        hi = lax.min(lo + per_w, full_blocks)
        iota = lax.iota(jnp.int32, 16)
        dstbase = [(iota + 16 * t) * D for t in range(8)]

        def do_block(r0, n_r):
            r0 = pl.multiple_of(r0, 128)
            for ct in range(D // 8):
                pltpu.async_copy(
                    tab_t.at[pl.ds(8 * ct, 8), pl.ds(r0, n_r)],
                    src_v.at[pl.ds(8 * ct, 8), pl.ds(0, n_r)],
                    sem,
                ).wait()

            @plsc.parallel_loop(0, D, unroll=4)
            def tr(c):
                for t in range(n_r // 16):
                    v = src_v[c, pl.ds(16 * t, 16)]
                    plsc.store_scatter(dst_v, [dstbase[t] + c], v)

            pltpu.sync_copy(
                dst_v.at[pl.ds(0, n_r * D)],
                out.at[pl.ds(pl.multiple_of(r0 * D, 128 * D), n_r * D)])

        def body(j, carry):
            do_block(j * 128, 128)
            return carry

        lax.fori_loop(lo, hi, body, None)
        if tail:
            @pl.when(w == NW - 1)
            def _():
                do_block(full_blocks * 128, tail)

    return detranspose


def _make_gather(n_lookups: int):
    per_w = n_lookups // NW
    subs_per_w = per_w // SUB
    n_chunks = per_w // CHUNK

    @functools.partial(
        pl.kernel,
        out_type=jax.ShapeDtypeStruct((n_lookups, D), jnp.float32),
        mesh=plsc.VectorSubcoreMesh(**_MESH),
        scratch_types=[
            pltpu.VMEM((subs_per_w, SUB), jnp.int32),
            pltpu.VMEM((NBUF, CHUNK, D), jnp.float32),
            pltpu.SemaphoreType.DMA((NBUF,)),
            pltpu.SemaphoreType.DMA((NBUF,)),
        ],
        compiler_params=pltpu.CompilerParams(use_tc_tiling_on_sc=False),
    )
    def gather_k(idx_hbm, table_hbm, out_hbm, idx_v, bufs, gsem, ssem):
        w = _wid()
        base = w * per_w
        pltpu.sync_copy(idx_hbm.at[pl.ds(w * subs_per_w, subs_per_w)], idx_v)

        def gather(g, b):
            return [
                pltpu.make_async_copy(
                    table_hbm.at[idx_v.at[g * GPC + j]],
                    bufs.at[b].at[pl.ds(j * SUB, SUB)],
                    gsem.at[b],
                )
                for j in range(GPC)
            ]

        def scatter(g, b):
            return pltpu.make_async_copy(
                bufs.at[b], out_hbm.at[pl.ds(base + g * CHUNK, CHUNK)],
                ssem.at[b],
            )

        for b in range(NBUF - 1):
            for cp in gather(b, b):
                cp.start()

        def outer(k, carry):
            g0 = k * NBUF
            for b in range(NBUF):
                g = g0 + b
                for cp in gather(g, b):
                    cp.wait()
                scatter(g, b).start()
                pb = (b - 1) % NBUF

                @pl.when(g > 0)
                def _():
                    scatter(g - 1, pb).wait()

                @pl.when(g + NBUF - 1 < n_chunks)
                def _():
                    for cp in gather(g + NBUF - 1, pb):
                        cp.start()

            return carry

        lax.fori_loop(0, n_chunks // NBUF, outer, None)
        scatter(n_chunks - 1, (n_chunks - 1) % NBUF).wait()

    return gather_k


def _make_retile(n_lookups: int, n_l: int, n_b: int):
    per_w = n_lookups // NW
    blocks_per_w = per_w // BW

    @functools.partial(
        pl.kernel,
        out_type=jax.ShapeDtypeStruct((n_l, D, n_b), jnp.float32),
        mesh=plsc.VectorSubcoreMesh(**_MESH),
        scratch_types=[
            pltpu.VMEM((BW * D,), jnp.float32),
            pltpu.VMEM((D, BW), jnp.float32),
            pltpu.SemaphoreType.DMA,
        ],
        compiler_params=pltpu.CompilerParams(
            use_tc_tiling_on_sc=True, needs_layout_passes=False,
            disable_bounds_checks=True),
    )
    def retile(rows_hbm, out, iv, ov, sem):
        w = _wid()
        n_base = w * per_w
        iota = lax.iota(jnp.int32, 16)
        srcbase = [(iota + 16 * t) * D for t in range(BW // 16)]

        def blk(k, carry):
            n0 = n_base + k * BW
            l = lax.shift_right_logical(n0, 12)
            b0 = pl.multiple_of(n0 & (n_b - 1), BW)
            pltpu.sync_copy(
                rows_hbm.at[pl.ds(pl.multiple_of(n0 * D, BW * D), BW * D)], iv)

            @plsc.parallel_loop(0, D, unroll=4)
            def tc(c):
                for t in range(BW // 16):
                    v = plsc.load_gather(iv, [srcbase[t] + c])
                    ov[c, pl.ds(16 * t, 16)] = v * SCALE
            for ct in range(D // 8):
                pltpu.async_copy(
                    ov.at[pl.ds(8 * ct, 8), :],
                    out.at[l, pl.ds(8 * ct, 8), pl.ds(b0, BW)],
                    sem,
                ).wait()
            return carry

        lax.fori_loop(0, blocks_per_w, blk, None)

    return retile


def kernel(input_, table):
    l, b = input_.shape
    n = l * b
    v, d = table.shape
    tab_t = table.T
    tab_lin = _make_detranspose(v)(tab_t).reshape(v, d)
    idx2d = input_.reshape(n // SUB, SUB)
    rows = _make_gather(n)(idx2d, tab_lin)
    out_t = _make_retile(n, l, b)(rows.reshape(n * d))
    return out_t.transpose(0, 2, 1)


# speedup vs baseline: 1.4007x; 1.4007x over previous
import functools

import jax
import jax.numpy as jnp
from jax import lax
from jax.experimental import pallas as pl
from jax.experimental.pallas import tpu as pltpu
from jax.experimental.pallas import tpu_sc as plsc

D = 64
SCALE = 8.0
SUB = 128
GPC = 2
CHUNK = SUB * GPC
NBUF = 4
BW = 256

_info = plsc.get_sparse_core_info()
_NC, _NS = _info.num_cores, _info.num_subcores
NW = _NC * _NS

_MESH = dict(core_axis_name="c", subcore_axis_name="s")


def _wid():
    return lax.axis_index("s") * _NC + lax.axis_index("c")


def _make_detranspose(n_rows: int):
    full_blocks = n_rows // 128
    tail = n_rows - full_blocks * 128
    per_w = (full_blocks + NW - 1) // NW

    @functools.partial(
        pl.kernel,
        out_type=jax.ShapeDtypeStruct((n_rows * D,), jnp.float32),
        mesh=plsc.VectorSubcoreMesh(**_MESH),
        scratch_types=[
            pltpu.VMEM((D, 128), jnp.float32),
            pltpu.VMEM((128 * D,), jnp.float32),
            pltpu.SemaphoreType.DMA,
        ],
        compiler_params=pltpu.CompilerParams(
            use_tc_tiling_on_sc=True, needs_layout_passes=False,
            disable_bounds_checks=True),
    )
    def detranspose(tab_t, out, src_v, dst_v, sem):
        w = _wid()
        lo = w * per_w
        hi = lax.min(lo + per_w, full_blocks)
        iota = lax.iota(jnp.int32, 16)
        dstbase = [(iota + 16 * t) * D for t in range(8)]

        def do_block(r0, n_r):
            r0 = pl.multiple_of(r0, 128)
            cps = [pltpu.async_copy(
                tab_t.at[pl.ds(8 * ct, 8), pl.ds(r0, n_r)],
                src_v.at[pl.ds(8 * ct, 8), pl.ds(0, n_r)],
                sem) for ct in range(D // 8)]
            for cp in cps:
                cp.wait()

            @plsc.parallel_loop(0, D, unroll=4)
            def tr(c):
                for t in range(n_r // 16):
                    v = src_v[c, pl.ds(16 * t, 16)]
                    plsc.store_scatter(dst_v, [dstbase[t] + c], v)

            pltpu.sync_copy(
                dst_v.at[pl.ds(0, n_r * D)],
                out.at[pl.ds(pl.multiple_of(r0 * D, 128 * D), n_r * D)])

        def body(j, carry):
            do_block(j * 128, 128)
            return carry

        lax.fori_loop(lo, hi, body, None)
        if tail:
            @pl.when(w == NW - 1)
            def _():
                do_block(full_blocks * 128, tail)

    return detranspose


def _make_gather(n_lookups: int):
    per_w = n_lookups // NW
    subs_per_w = per_w // SUB
    n_chunks = per_w // CHUNK

    @functools.partial(
        pl.kernel,
        out_type=jax.ShapeDtypeStruct((n_lookups, D), jnp.float32),
        mesh=plsc.VectorSubcoreMesh(**_MESH),
        scratch_types=[
            pltpu.VMEM((subs_per_w, SUB), jnp.int32),
            pltpu.VMEM((NBUF, CHUNK, D), jnp.float32),
            pltpu.SemaphoreType.DMA((NBUF,)),
            pltpu.SemaphoreType.DMA((NBUF,)),
        ],
        compiler_params=pltpu.CompilerParams(use_tc_tiling_on_sc=False),
    )
    def gather_k(idx_hbm, table_hbm, out_hbm, idx_v, bufs, gsem, ssem):
        w = _wid()
        base = w * per_w
        pltpu.sync_copy(idx_hbm.at[pl.ds(w * subs_per_w, subs_per_w)], idx_v)

        def gather(g, b):
            return [
                pltpu.make_async_copy(
                    table_hbm.at[idx_v.at[g * GPC + j]],
                    bufs.at[b].at[pl.ds(j * SUB, SUB)],
                    gsem.at[b],
                )
                for j in range(GPC)
            ]

        def scatter(g, b):
            return pltpu.make_async_copy(
                bufs.at[b], out_hbm.at[pl.ds(base + g * CHUNK, CHUNK)],
                ssem.at[b],
            )

        for b in range(NBUF - 1):
            for cp in gather(b, b):
                cp.start()

        def outer(k, carry):
            g0 = k * NBUF
            for b in range(NBUF):
                g = g0 + b
                for cp in gather(g, b):
                    cp.wait()
                scatter(g, b).start()
                pb = (b - 1) % NBUF

                @pl.when(g > 0)
                def _():
                    scatter(g - 1, pb).wait()

                @pl.when(g + NBUF - 1 < n_chunks)
                def _():
                    for cp in gather(g + NBUF - 1, pb):
                        cp.start()

            return carry

        lax.fori_loop(0, n_chunks // NBUF, outer, None)
        scatter(n_chunks - 1, (n_chunks - 1) % NBUF).wait()

    return gather_k


def _make_retile(n_lookups: int, n_l: int, n_b: int):
    per_w = n_lookups // NW
    blocks_per_w = per_w // BW

    @functools.partial(
        pl.kernel,
        out_type=jax.ShapeDtypeStruct((n_l, D, n_b), jnp.float32),
        mesh=plsc.VectorSubcoreMesh(**_MESH),
        scratch_types=[
            pltpu.VMEM((BW * D,), jnp.float32),
            pltpu.VMEM((D, BW), jnp.float32),
            pltpu.SemaphoreType.DMA,
        ],
        compiler_params=pltpu.CompilerParams(
            use_tc_tiling_on_sc=True, needs_layout_passes=False,
            disable_bounds_checks=True),
    )
    def retile(rows_hbm, out, iv, ov, sem):
        w = _wid()
        n_base = w * per_w
        iota = lax.iota(jnp.int32, 16)
        srcbase = [(iota + 16 * t) * D for t in range(BW // 16)]

        def blk(k, carry):
            n0 = n_base + k * BW
            l = lax.shift_right_logical(n0, 12)
            b0 = pl.multiple_of(n0 & (n_b - 1), BW)
            pltpu.sync_copy(
                rows_hbm.at[pl.ds(pl.multiple_of(n0 * D, BW * D), BW * D)], iv)

            @plsc.parallel_loop(0, D, unroll=4)
            def tc(c):
                for t in range(BW // 16):
                    v = plsc.load_gather(iv, [srcbase[t] + c])
                    ov[c, pl.ds(16 * t, 16)] = v * SCALE
            cps = [pltpu.async_copy(
                ov.at[pl.ds(8 * ct, 8), :],
                out.at[l, pl.ds(8 * ct, 8), pl.ds(b0, BW)],
                sem) for ct in range(D // 8)]
            for cp in cps:
                cp.wait()
            return carry

        lax.fori_loop(0, blocks_per_w, blk, None)

    return retile


def kernel(input_, table):
    l, b = input_.shape
    n = l * b
    v, d = table.shape
    tab_t = table.T
    tab_lin = _make_detranspose(v)(tab_t).reshape(v, d)
    idx2d = input_.reshape(n // SUB, SUB)
    rows = _make_gather(n)(idx2d, tab_lin)
    out_t = _make_retile(n, l, b)(rows.reshape(n * d))
    return out_t.transpose(0, 2, 1)


# speedup vs baseline: 1.6691x; 1.1916x over previous
import functools

import jax
import jax.numpy as jnp
from jax import lax
from jax.experimental import pallas as pl
from jax.experimental.pallas import tpu as pltpu
from jax.experimental.pallas import tpu_sc as plsc

D = 64
SCALE = 8.0
SUB = 128
GPC = 2
CHUNK = SUB * GPC
NBUF = 4
BW = 256

_info = plsc.get_sparse_core_info()
_NC, _NS = _info.num_cores, _info.num_subcores
NW = _NC * _NS

_MESH = dict(core_axis_name="c", subcore_axis_name="s")


def _wid():
    return lax.axis_index("s") * _NC + lax.axis_index("c")


def _make_detranspose(n_rows: int):
    AW = 256
    full_blocks = n_rows // AW
    tail = n_rows - full_blocks * AW
    per_w = (full_blocks + NW - 1) // NW

    @functools.partial(
        pl.kernel,
        out_type=jax.ShapeDtypeStruct((n_rows * D,), jnp.float32),
        mesh=plsc.VectorSubcoreMesh(**_MESH),
        scratch_types=[
            pltpu.VMEM((D, AW), jnp.float32),
            pltpu.VMEM((D, AW), jnp.float32),
            pltpu.VMEM((AW * D,), jnp.float32),
            pltpu.VMEM((AW * D,), jnp.float32),
            pltpu.SemaphoreType.DMA((2,)),
            pltpu.SemaphoreType.DMA((2,)),
        ],
        compiler_params=pltpu.CompilerParams(
            use_tc_tiling_on_sc=True, needs_layout_passes=False,
            disable_bounds_checks=True),
    )
    def detranspose(tab_t, tail_in, out, src_v0, src_v1, dst_v0, dst_v1, gsem, ssem):
        srcs, dsts = [src_v0, src_v1], [dst_v0, dst_v1]
        w = _wid()
        lo = w * per_w
        hi = lax.min(lo + per_w, full_blocks)
        n = hi - lo
        iota = lax.iota(jnp.int32, 16)
        dstbase = [(iota + 16 * t) * D for t in range(AW // 16)]

        def in_copies(j, b):
            r0 = pl.multiple_of(j * AW, AW)
            return [pltpu.make_async_copy(
                tab_t.at[pl.ds(8 * ct, 8), pl.ds(r0, AW)],
                srcs[b].at[pl.ds(8 * ct, 8), :],
                gsem.at[b]) for ct in range(D // 8)]

        def out_copy(j, b):
            r0 = pl.multiple_of(j * AW, AW)
            return pltpu.make_async_copy(
                dsts[b],
                out.at[pl.ds(pl.multiple_of(r0 * D, AW * D), AW * D)],
                ssem.at[b])

        for cp in in_copies(lo, 0):
            cp.start()

        def body(k, carry):
            b = k & 1
            for b2 in range(2):
                @pl.when(b == b2)
                def _(b2=b2):
                    @pl.when(k + 1 < n)
                    def _():
                        for cp in in_copies(lo + k + 1, 1 - b2):
                            cp.start()

                    for cp in in_copies(lo + k, b2):
                        cp.wait()

                    @pl.when(k >= 2)
                    def _():
                        out_copy(lo + k - 2, b2).wait()

                    @plsc.parallel_loop(0, D, unroll=4)
                    def tr(c):
                        for t in range(AW // 16):
                            v = srcs[b2][c, pl.ds(16 * t, 16)]
                            plsc.store_scatter(dsts[b2],
                                               [dstbase[t] + c], v)

                    out_copy(lo + k, b2).start()
            return carry

        lax.fori_loop(0, n, body, None)

        for b2 in range(2):
            @pl.when((n >= 2) & ((n & 1) == b2))
            def _(b2=b2):
                out_copy(lo + n - 2, b2).wait()

            @pl.when(((n - 1) & 1) == b2)
            def _(b2=b2):
                out_copy(lo + n - 1, b2).wait()

        if tail:
            @pl.when(w == NW - 1)
            def _():
                r0 = full_blocks * AW
                pltpu.sync_copy(tail_in, src_v0.at[:, pl.ds(0, 128)])

                @plsc.parallel_loop(0, D, unroll=4)
                def tr(c):
                    for t in range(tail // 16):
                        v = src_v0[c, pl.ds(16 * t, 16)]
                        plsc.store_scatter(dst_v0, [dstbase[t] + c], v)

                pltpu.sync_copy(dst_v0.at[pl.ds(0, tail * D)],
                                out.at[pl.ds(r0 * D, tail * D)])

    return detranspose


def _make_gather(n_lookups: int):
    per_w = n_lookups // NW
    subs_per_w = per_w // SUB
    n_chunks = per_w // CHUNK

    @functools.partial(
        pl.kernel,
        out_type=jax.ShapeDtypeStruct((n_lookups, D), jnp.float32),
        mesh=plsc.VectorSubcoreMesh(**_MESH),
        scratch_types=[
            pltpu.VMEM((subs_per_w, SUB), jnp.int32),
            pltpu.VMEM((NBUF, CHUNK, D), jnp.float32),
            pltpu.SemaphoreType.DMA((NBUF,)),
            pltpu.SemaphoreType.DMA((NBUF,)),
        ],
        compiler_params=pltpu.CompilerParams(use_tc_tiling_on_sc=False),
    )
    def gather_k(idx_hbm, table_hbm, out_hbm, idx_v, bufs, gsem, ssem):
        w = _wid()
        base = w * per_w
        pltpu.sync_copy(idx_hbm.at[pl.ds(w * subs_per_w, subs_per_w)], idx_v)

        def gather(g, b):
            return [
                pltpu.make_async_copy(
                    table_hbm.at[idx_v.at[g * GPC + j]],
                    bufs.at[b].at[pl.ds(j * SUB, SUB)],
                    gsem.at[b],
                )
                for j in range(GPC)
            ]

        def scatter(g, b):
            return pltpu.make_async_copy(
                bufs.at[b], out_hbm.at[pl.ds(base + g * CHUNK, CHUNK)],
                ssem.at[b],
            )

        for b in range(NBUF - 1):
            for cp in gather(b, b):
                cp.start()

        def outer(k, carry):
            g0 = k * NBUF
            for b in range(NBUF):
                g = g0 + b
                for cp in gather(g, b):
                    cp.wait()
                scatter(g, b).start()
                pb = (b - 1) % NBUF

                @pl.when(g > 0)
                def _():
                    scatter(g - 1, pb).wait()

                @pl.when(g + NBUF - 1 < n_chunks)
                def _():
                    for cp in gather(g + NBUF - 1, pb):
                        cp.start()

            return carry

        lax.fori_loop(0, n_chunks // NBUF, outer, None)
        scatter(n_chunks - 1, (n_chunks - 1) % NBUF).wait()

    return gather_k


def _make_retile(n_lookups: int, n_l: int, n_b: int):
    per_w = n_lookups // NW
    blocks_per_w = per_w // BW

    @functools.partial(
        pl.kernel,
        out_type=jax.ShapeDtypeStruct((n_l, D, n_b), jnp.float32),
        mesh=plsc.VectorSubcoreMesh(**_MESH),
        scratch_types=[
            pltpu.VMEM((BW * D,), jnp.float32),
            pltpu.VMEM((D, BW), jnp.float32),
            pltpu.SemaphoreType.DMA,
        ],
        compiler_params=pltpu.CompilerParams(
            use_tc_tiling_on_sc=True, needs_layout_passes=False,
            disable_bounds_checks=True),
    )
    def retile(rows_hbm, out, iv, ov, sem):
        w = _wid()
        n_base = w * per_w
        iota = lax.iota(jnp.int32, 16)
        srcbase = [(iota + 16 * t) * D for t in range(BW // 16)]

        def blk(k, carry):
            n0 = n_base + k * BW
            l = lax.shift_right_logical(n0, 12)
            b0 = pl.multiple_of(n0 & (n_b - 1), BW)
            pltpu.sync_copy(
                rows_hbm.at[pl.ds(pl.multiple_of(n0 * D, BW * D), BW * D)], iv)

            @plsc.parallel_loop(0, D, unroll=4)
            def tc(c):
                for t in range(BW // 16):
                    v = plsc.load_gather(iv, [srcbase[t] + c])
                    ov[c, pl.ds(16 * t, 16)] = v * SCALE
            cps = [pltpu.async_copy(
                ov.at[pl.ds(8 * ct, 8), :],
                out.at[l, pl.ds(8 * ct, 8), pl.ds(b0, BW)],
                sem) for ct in range(D // 8)]
            for cp in cps:
                cp.wait()
            return carry

        lax.fori_loop(0, blocks_per_w, blk, None)

    return retile


def kernel(input_, table):
    l, b = input_.shape
    n = l * b
    v, d = table.shape
    tab_t = table.T
    tail_start = (v // 256) * 256
    tail_pad = jnp.pad(tab_t[:, tail_start:], ((0, 0), (0, 128 - (v - tail_start))))
    tab_lin = _make_detranspose(v)(tab_t, tail_pad).reshape(v, d)
    idx2d = input_.reshape(n // SUB, SUB)
    rows = _make_gather(n)(idx2d, tab_lin)
    out_t = _make_retile(n, l, b)(rows.reshape(n * d))
    return out_t.transpose(0, 2, 1)


# speedup vs baseline: 2.6038x; 1.5600x over previous
import functools

import jax
import jax.numpy as jnp
from jax import lax
from jax.experimental import pallas as pl
from jax.experimental.pallas import tpu as pltpu
from jax.experimental.pallas import tpu_sc as plsc

D = 64
SCALE = 8.0
SUB = 128
GPC = 2
C = SUB * GPC
NBUF = 4


def _make_lookup(n_rows: int):
    info = plsc.get_sparse_core_info()
    nc, ns = info.num_cores, info.num_subcores
    nw = nc * ns
    per_w = n_rows // nw
    subs_per_w = per_w // SUB
    n_chunks = per_w // C

    mesh = plsc.VectorSubcoreMesh(core_axis_name="c", subcore_axis_name="s")

    @functools.partial(
        pl.kernel,
        out_type=jax.ShapeDtypeStruct((n_rows, D), jnp.float32),
        mesh=mesh,
        scratch_types=[
            pltpu.VMEM((subs_per_w, SUB), jnp.int32),
            pltpu.VMEM((NBUF, C, D), jnp.float32),
            pltpu.SemaphoreType.DMA((NBUF,)),
            pltpu.SemaphoreType.DMA((NBUF,)),
        ],
        compiler_params=pltpu.CompilerParams(use_tc_tiling_on_sc=False),
    )
    def lookup(idx_hbm, table_hbm, out_hbm, idx_v, bufs, gsem, ssem):
        wid = lax.axis_index("s") * nc + lax.axis_index("c")
        base = wid * per_w
        pltpu.sync_copy(idx_hbm.at[pl.ds(wid * subs_per_w, subs_per_w)], idx_v)

        def gather(g, b):
            return [
                pltpu.make_async_copy(
                    table_hbm.at[idx_v.at[g * GPC + j]],
                    bufs.at[b].at[pl.ds(j * SUB, SUB)],
                    gsem.at[b],
                )
                for j in range(GPC)
            ]

        def scatter(g, b):
            return pltpu.make_async_copy(
                bufs.at[b], out_hbm.at[pl.ds(base + g * C, C)], ssem.at[b]
            )

        for b in range(NBUF - 1):
            for cp in gather(b, b):
                cp.start()

        def outer(k, carry):
            g0 = k * NBUF
            for b in range(NBUF):
                g = g0 + b
                for cp in gather(g, b):
                    cp.wait()

                def scale_row(i, c2):
                    for t in range(D // 16):
                        sl = pl.ds(t * 16, 16)
                        bufs[b, i, sl] = bufs[b, i, sl] * SCALE
                    return c2

                lax.fori_loop(0, C, scale_row, None)
                scatter(g, b).start()

                pb = (b - 1) % NBUF

                @pl.when(g > 0)
                def _():
                    scatter(g - 1, pb).wait()

                @pl.when(g + NBUF - 1 < n_chunks)
                def _():
                    for cp in gather(g + NBUF - 1, pb):
                        cp.start()

            return carry

        lax.fori_loop(0, n_chunks // NBUF, outer, None)
        scatter(n_chunks - 1, (n_chunks - 1) % NBUF).wait()

    return lookup


def kernel(input_, table):
    l, b = input_.shape
    n = l * b
    idx2d = input_.reshape(n // SUB, SUB)
    out = _make_lookup(n)(idx2d, table)
    return out.reshape(l, b, D)
